# baseline (speedup 1.0000x reference)
import jax
import jax.numpy as jnp
from jax.experimental import pallas as pl


_HB = 32


def _body(x_ref, row_ref, col_ref, o_ref):
    x = x_ref[...]
    row = row_ref[...]
    col = col_ref[...]
    hb = row.shape[0]
    sub = col.shape[0]
    rowpat = jnp.broadcast_to(row[:, None, :, :], (hb, sub // 24, 24, 128))
    rowpat = rowpat.reshape(hb, sub, 128)
    o_ref[...] = x + (rowpat + col[None, :, :])[None]


def kernel(inputs, row_embeddings, col_embeddings):
    b, h, w, c = inputs.shape
    hb = _HB
    wc = w * c
    sub = wc // 128
    x = inputs.reshape(b, h, sub, 128)
    row24 = jnp.tile(row_embeddings, (1, (24 * 128) // c)).reshape(h, 24, 128)
    col2d = col_embeddings.reshape(sub, 128)
    grid = (h // hb, b)
    out = pl.pallas_call(
        _body,
        grid=grid,
        in_specs=[
            pl.BlockSpec((1, hb, sub, 128), lambda hi, bi: (bi, hi, 0, 0)),
            pl.BlockSpec((hb, 24, 128), lambda hi, bi: (hi, 0, 0)),
            pl.BlockSpec((sub, 128), lambda hi, bi: (0, 0)),
        ],
        out_specs=pl.BlockSpec((1, hb, sub, 128), lambda hi, bi: (bi, hi, 0, 0)),
        out_shape=jax.ShapeDtypeStruct((b, h, sub, 128), inputs.dtype),
    )(x, row24, col2d)
    return out.reshape(b, h, w, c)


# speedup vs baseline: 1.2639x; 1.2639x over previous
import jax
import jax.numpy as jnp
from jax.experimental import pallas as pl


_HB = 8


def _body(x_ref, row_ref, col_ref, o_ref):
    x = x_ref[...]
    row = row_ref[...]
    col = col_ref[...]
    pos = row[:, None, :] + col[None, :, :]
    o_ref[...] = x + pos[None, :, :, :]


def kernel(inputs, row_embeddings, col_embeddings):
    b, h, w, c = inputs.shape
    hb = _HB
    grid = (b, h // hb)
    return pl.pallas_call(
        _body,
        grid=grid,
        in_specs=[
            pl.BlockSpec((1, hb, w, c), lambda bi, hi: (bi, hi, 0, 0)),
            pl.BlockSpec((hb, c), lambda bi, hi: (hi, 0)),
            pl.BlockSpec((w, c), lambda bi, hi: (0, 0)),
        ],
        out_specs=pl.BlockSpec((1, hb, w, c), lambda bi, hi: (bi, hi, 0, 0)),
        out_shape=jax.ShapeDtypeStruct((b, h, w, c), inputs.dtype),
    )(inputs, row_embeddings, col_embeddings)


# speedup vs baseline: 1.2892x; 1.0200x over previous
import jax
import jax.numpy as jnp
from jax.experimental import pallas as pl
from jax.experimental.pallas import tpu as pltpu


_HB = 8
_DEPTH = 8


def _body(x_hbm, row_ref, col_ref, o_hbm, xb, ob, posb, in_sems, out_sems):
    nb = x_hbm.shape[0]
    h = x_hbm.shape[1]
    hb = posb.shape[0]
    n = (h // hb) * nb
    i = pl.program_id(0)
    hi = i // nb
    bi = i % nb
    slot = jax.lax.rem(i, _DEPTH)

    def read(step):
        s_hi = step // nb
        s_bi = step % nb
        s_slot = jax.lax.rem(step, _DEPTH)
        pltpu.make_async_copy(
            x_hbm.at[s_bi, pl.ds(s_hi * hb, hb)],
            xb.at[s_slot],
            in_sems.at[s_slot],
        ).start()

    @pl.when(i == 0)
    def _prologue():
        for d in range(_DEPTH):
            read(jnp.int32(d))

    pltpu.make_async_copy(
        x_hbm.at[bi, pl.ds(hi * hb, hb)], xb.at[slot], in_sems.at[slot]
    ).wait()

    @pl.when(bi == 0)
    def _pos():
        row = row_ref[pl.ds(hi * hb, hb), :]
        col = col_ref[...]
        posb[...] = row[:, None, :] + col[None, :, :]

    @pl.when(i >= _DEPTH)
    def _drain_out():
        pltpu.make_async_copy(
            ob.at[slot], o_hbm.at[bi, pl.ds(hi * hb, hb)], out_sems.at[slot]
        ).wait()

    ob[slot] = xb[slot] + posb[...]

    pltpu.make_async_copy(
        ob.at[slot], o_hbm.at[bi, pl.ds(hi * hb, hb)], out_sems.at[slot]
    ).start()

    @pl.when(i + _DEPTH < n)
    def _next_read():
        read(i + _DEPTH)

    @pl.when(i == n - 1)
    def _epilogue():
        for d in range(_DEPTH):
            step = n - _DEPTH + d
            s_hi = step // nb
            s_bi = step % nb
            pltpu.make_async_copy(
                ob.at[d], o_hbm.at[s_bi, pl.ds(s_hi * hb, hb)], out_sems.at[d]
            ).wait()


def kernel(inputs, row_embeddings, col_embeddings):
    b, h, w, c = inputs.shape
    hb = _HB
    n = (h // hb) * b
    return pl.pallas_call(
        _body,
        grid=(n,),
        in_specs=[
            pl.BlockSpec(memory_space=pltpu.MemorySpace.HBM),
            pl.BlockSpec((h, c), lambda i: (0, 0)),
            pl.BlockSpec((w, c), lambda i: (0, 0)),
        ],
        out_specs=pl.BlockSpec(memory_space=pltpu.MemorySpace.HBM),
        out_shape=jax.ShapeDtypeStruct((b, h, w, c), inputs.dtype),
        scratch_shapes=[
            pltpu.VMEM((_DEPTH, hb, w, c), inputs.dtype),
            pltpu.VMEM((_DEPTH, hb, w, c), inputs.dtype),
            pltpu.VMEM((hb, w, c), inputs.dtype),
            pltpu.SemaphoreType.DMA((_DEPTH,)),
            pltpu.SemaphoreType.DMA((_DEPTH,)),
        ],
        compiler_params=pltpu.CompilerParams(
            dimension_semantics=("arbitrary",),
        ),
    )(inputs, row_embeddings, col_embeddings)
